# Initial kernel scaffold; baseline (speedup 1.0000x reference)
import jax
import jax.numpy as jnp
from jax.experimental import pallas as pl


def kernel(x, adj, W, a_self, a_neigh, b):
    raise NotImplementedError("write your pallas kernel here")


# speedup vs baseline: 1.6695x; 1.6695x over previous
import functools

import jax
import jax.numpy as jnp
from jax import lax
from jax.experimental import pallas as pl
from jax.experimental.pallas import tpu as pltpu

_L = 2
_H = 2
_F_IN = 128
_F_OUT = 64
_N = 4096
_R = 256
_NB = _N // _R
_NEG = -10e9


def _layer_body(h_ref, adj_ref, w_ref, as_ref, an_ref, b_ref,
                out_ref, adj8_ref, feat_scr, *, first_layer):
    i = pl.program_id(1)

    a = adj_ref[0]
    if first_layer:
        amask = a > 0.0
        adj8_ref[0] = amask.astype(jnp.int8)
    else:
        amask = a.astype(jnp.float32) > 0.0

    @pl.when(i == 0)
    def _compute_feats():
        hfull = h_ref[0]
        for hd in range(_H):
            feat_scr[hd] = jnp.dot(hfull, w_ref[hd],
                                   preferred_element_type=jnp.float32)

    outs = []
    for hd in range(_H):
        feat = feat_scr[hd]
        feat_blk = feat_scr[hd, pl.ds(i * _R, _R), :]
        asl = as_ref[hd].reshape(1, _F_OUT)
        anl = an_ref[hd].reshape(1, _F_OUT)
        fs = lax.dot_general(feat_blk, asl, (((1,), (1,)), ((), ())),
                             preferred_element_type=jnp.float32)
        fn = lax.dot_general(anl, feat, (((1,), (1,)), ((), ())),
                             preferred_element_type=jnp.float32)
        logits = fs + fn
        logits = jnp.where(logits >= 0.0, logits, 0.2 * logits)
        masked = jnp.where(amask, logits, logits + _NEG)
        rowmax = jnp.max(masked, axis=1, keepdims=True)
        p = jnp.exp(masked - rowmax)
        rowsum = jnp.sum(p, axis=1, keepdims=True)
        o = jnp.dot(p, feat, preferred_element_type=jnp.float32)
        outs.append(o / rowsum + b_ref[hd].reshape(1, _F_OUT))

    out = jnp.concatenate(outs, axis=-1)
    out_ref[0] = jnp.where(out > 0.0, out, jnp.exp(out) - 1.0)


def _run_layer(h, adj, Wl, asl, anl, bl, *, first_layer):
    B = h.shape[0]
    grid = (B, _NB)
    in_specs = [
        pl.BlockSpec((1, _N, _F_IN), lambda g, i: (g, 0, 0)),
        pl.BlockSpec((1, _R, _N), lambda g, i: (g, i, 0)),
        pl.BlockSpec((_H, _F_IN, _F_OUT), lambda g, i: (0, 0, 0)),
        pl.BlockSpec((_H, _F_OUT), lambda g, i: (0, 0)),
        pl.BlockSpec((_H, _F_OUT), lambda g, i: (0, 0)),
        pl.BlockSpec((_H, _F_OUT), lambda g, i: (0, 0)),
    ]
    out_spec_h = pl.BlockSpec((1, _R, _H * _F_OUT), lambda g, i: (g, i, 0))
    out_shape_h = jax.ShapeDtypeStruct((B, _N, _H * _F_OUT), jnp.float32)
    if first_layer:
        out_specs = [out_spec_h,
                     pl.BlockSpec((1, _R, _N), lambda g, i: (g, i, 0))]
        out_shapes = [out_shape_h,
                      jax.ShapeDtypeStruct((B, _N, _N), jnp.int8)]
    else:
        out_specs = [out_spec_h]
        out_shapes = [out_shape_h]

    body = functools.partial(_layer_body, first_layer=first_layer)
    if not first_layer:
        def body(h_ref, adj_ref, w_ref, as_ref, an_ref, b_ref, out_ref,
                 feat_scr):
            _layer_body(h_ref, adj_ref, w_ref, as_ref, an_ref, b_ref,
                        out_ref, None, feat_scr, first_layer=False)

    return pl.pallas_call(
        body,
        grid=grid,
        in_specs=in_specs,
        out_specs=out_specs,
        out_shape=out_shapes,
        scratch_shapes=[pltpu.VMEM((_H, _N, _F_OUT), jnp.float32)],
    )(h, adj, Wl, asl, anl, bl)


def kernel(x, adj, W, a_self, a_neigh, b):
    h = x
    h, adj8 = _run_layer(h, adj, W[0], a_self[0], a_neigh[0], b[0],
                         first_layer=True)
    for l in range(1, _L):
        (h,) = _run_layer(h, adj8, W[l], a_self[l], a_neigh[l], b[l],
                          first_layer=False)
    return h


# speedup vs baseline: 2.4358x; 1.4590x over previous
import jax
import jax.numpy as jnp
from jax import lax
from jax.experimental import pallas as pl
from jax.experimental.pallas import tpu as pltpu

_L = 2
_H = 2
_F_IN = 128
_F_OUT = 64
_N = 4096
_R = 256
_NB = _N // _R


def _layer_body(h_ref, adj_ref, w_ref, as_ref, an_ref, b_ref,
                out_ref, adj8_ref, feat_scr, ext_scr, en_scr, fnmax_scr,
                *, first_layer):
    i = pl.program_id(1)

    a = adj_ref[0]
    if first_layer:
        af = a
        adj8_ref[0] = (a > 0.0).astype(jnp.int8)
    else:
        af = a.astype(jnp.float32)

    @pl.when(i == 0)
    def _per_graph_prologue():
        hfull = h_ref[0]
        for hd in range(_H):
            feat = jnp.dot(hfull, w_ref[hd],
                           preferred_element_type=jnp.float32)
            feat_scr[hd] = feat
            ext_scr[hd] = feat.astype(jnp.bfloat16)
            anl = an_ref[hd]
            fn = lax.dot_general(anl, feat, (((1,), (1,)), ((), ())),
                                 preferred_element_type=jnp.float32)
            fnmax = jnp.max(fn)
            fnmax_scr[hd] = fnmax
            en_scr[hd, 0:1, :] = jnp.exp(fn - fnmax)
            en_scr[hd, 1:2, :] = jnp.exp(0.2 * (fn - fnmax))

    outs = []
    for hd in range(_H):
        feat_blk = feat_scr[hd, pl.ds(i * _R, _R), :]
        fs = jnp.dot(feat_blk, as_ref[hd],
                     preferred_element_type=jnp.float32)
        fnmax = fnmax_scr[hd]
        xmax = fs + fnmax
        s = jnp.where(xmax >= 0.0, xmax, 0.2 * xmax)
        ea = jnp.exp(xmax - s)
        eb = jnp.exp(0.2 * xmax - s)
        en = en_scr[hd, 0:1, :]
        en2 = en_scr[hd, 1:2, :]
        p = af * jnp.maximum(ea * en, eb * en2)
        rowsum = jnp.sum(p, axis=1, keepdims=True)
        o = jnp.dot(p.astype(jnp.bfloat16), ext_scr[hd],
                    preferred_element_type=jnp.float32)
        outs.append(o / rowsum + b_ref[hd])

    out = jnp.concatenate(outs, axis=-1)
    out_ref[0] = jnp.where(out > 0.0, out, jnp.exp(out) - 1.0)


def _run_layer(h, adj, Wl, asl, anl, bl, *, first_layer):
    B = h.shape[0]
    grid = (B, _NB)
    in_specs = [
        pl.BlockSpec((1, _N, _F_IN), lambda g, i: (g, 0, 0)),
        pl.BlockSpec((1, _R, _N), lambda g, i: (g, i, 0)),
        pl.BlockSpec((_H, _F_IN, _F_OUT), lambda g, i: (0, 0, 0)),
        pl.BlockSpec((_H, _F_OUT, 1), lambda g, i: (0, 0, 0)),
        pl.BlockSpec((_H, 1, _F_OUT), lambda g, i: (0, 0, 0)),
        pl.BlockSpec((_H, 1, _F_OUT), lambda g, i: (0, 0, 0)),
    ]
    out_spec_h = pl.BlockSpec((1, _R, _H * _F_OUT), lambda g, i: (g, i, 0))
    out_shape_h = jax.ShapeDtypeStruct((B, _N, _H * _F_OUT), jnp.float32)
    if first_layer:
        out_specs = [out_spec_h,
                     pl.BlockSpec((1, _R, _N), lambda g, i: (g, i, 0))]
        out_shapes = [out_shape_h,
                      jax.ShapeDtypeStruct((B, _N, _N), jnp.int8)]

        def body(h_ref, adj_ref, w_ref, as_ref, an_ref, b_ref, out_ref,
                 adj8_ref, feat_scr, ext_scr, en_scr, fnmax_scr):
            _layer_body(h_ref, adj_ref, w_ref, as_ref, an_ref, b_ref,
                        out_ref, adj8_ref, feat_scr, ext_scr, en_scr,
                        fnmax_scr, first_layer=True)
    else:
        out_specs = [out_spec_h]
        out_shapes = [out_shape_h]

        def body(h_ref, adj_ref, w_ref, as_ref, an_ref, b_ref, out_ref,
                 feat_scr, ext_scr, en_scr, fnmax_scr):
            _layer_body(h_ref, adj_ref, w_ref, as_ref, an_ref, b_ref,
                        out_ref, None, feat_scr, ext_scr, en_scr,
                        fnmax_scr, first_layer=False)

    return pl.pallas_call(
        body,
        grid=grid,
        in_specs=in_specs,
        out_specs=out_specs,
        out_shape=out_shapes,
        scratch_shapes=[
            pltpu.VMEM((_H, _N, _F_OUT), jnp.float32),
            pltpu.VMEM((_H, _N, _F_OUT), jnp.bfloat16),
            pltpu.VMEM((_H, 2, _N), jnp.float32),
            pltpu.SMEM((_H,), jnp.float32),
        ],
    )(h, adj, Wl, asl[:, :, None], anl[:, None, :], bl[:, None, :])


def kernel(x, adj, W, a_self, a_neigh, b):
    h, adj8 = _run_layer(x, adj, W[0], a_self[0], a_neigh[0], b[0],
                         first_layer=True)
    for l in range(1, _L):
        (h,) = _run_layer(h, adj8, W[l], a_self[l], a_neigh[l], b[l],
                          first_layer=False)
    return h


# speedup vs baseline: 3.3395x; 1.3710x over previous
import jax
import jax.numpy as jnp
from jax import lax
from jax.experimental import pallas as pl
from jax.experimental.pallas import tpu as pltpu

_L = 2
_H = 2
_F_IN = 128
_F_OUT = 64
_N = 4096
_R = 256
_NB = _N // _R


def _layer_body(h_ref, adj_ref, w_ref, as_ref, an_ref, b_ref,
                out_ref, adjb_ref, feat_scr, ext_scr, en_scr, fnmax_scr,
                *, first_layer):
    i = pl.program_id(1)

    a = adj_ref[0]
    if first_layer:
        af = a.astype(jnp.bfloat16)
        adjb_ref[0] = af
    else:
        af = a

    @pl.when(i == 0)
    def _per_graph_prologue():
        hfull = h_ref[0]
        onescol = jnp.where(
            lax.broadcasted_iota(jnp.int32, (_N, _F_OUT), 1) == 0, 1.0, 0.0)
        for hd in range(_H):
            feat = jnp.dot(hfull, w_ref[hd],
                           preferred_element_type=jnp.float32)
            feat_scr[hd] = feat
            ext_scr[hd] = jnp.concatenate(
                [feat, onescol], axis=1).astype(jnp.bfloat16)
            anl = an_ref[hd]
            fn = lax.dot_general(anl, feat, (((1,), (1,)), ((), ())),
                                 preferred_element_type=jnp.float32)
            fnmax = jnp.max(fn)
            fnmax_scr[hd] = fnmax
            en_scr[hd, 0:1, :] = jnp.exp(fn - fnmax).astype(jnp.bfloat16)
            en_scr[hd, 1:2, :] = jnp.exp(
                0.2 * (fn - fnmax)).astype(jnp.bfloat16)

    outs = []
    for hd in range(_H):
        feat_blk = feat_scr[hd, pl.ds(i * _R, _R), :]
        fs = jnp.dot(feat_blk, as_ref[hd],
                     preferred_element_type=jnp.float32)
        fnmax = fnmax_scr[hd]
        xmax = fs + fnmax
        s = jnp.where(xmax >= 0.0, xmax, 0.2 * xmax)
        ea = jnp.exp(xmax - s).astype(jnp.bfloat16)
        eb = jnp.exp(0.2 * xmax - s).astype(jnp.bfloat16)
        en = en_scr[hd, 0:1, :]
        en2 = en_scr[hd, 1:2, :]
        p = af * jnp.maximum(ea * en, eb * en2)
        o2 = jnp.dot(p, ext_scr[hd],
                     preferred_element_type=jnp.float32)
        rowsum = o2[:, _F_OUT:_F_OUT + 1]
        outs.append(o2[:, 0:_F_OUT] / rowsum + b_ref[hd])

    out = jnp.concatenate(outs, axis=-1)
    out_ref[0] = jnp.where(out > 0.0, out, jnp.exp(out) - 1.0)


def _run_layer(h, adj, Wl, asl, anl, bl, *, first_layer):
    B = h.shape[0]
    grid = (B, _NB)
    in_specs = [
        pl.BlockSpec((1, _N, _F_IN), lambda g, i: (g, 0, 0)),
        pl.BlockSpec((1, _R, _N), lambda g, i: (g, i, 0)),
        pl.BlockSpec((_H, _F_IN, _F_OUT), lambda g, i: (0, 0, 0)),
        pl.BlockSpec((_H, _F_OUT, 1), lambda g, i: (0, 0, 0)),
        pl.BlockSpec((_H, 1, _F_OUT), lambda g, i: (0, 0, 0)),
        pl.BlockSpec((_H, 1, _F_OUT), lambda g, i: (0, 0, 0)),
    ]
    out_spec_h = pl.BlockSpec((1, _R, _H * _F_OUT), lambda g, i: (g, i, 0))
    out_shape_h = jax.ShapeDtypeStruct((B, _N, _H * _F_OUT), jnp.float32)
    if first_layer:
        out_specs = [out_spec_h,
                     pl.BlockSpec((1, _R, _N), lambda g, i: (g, i, 0))]
        out_shapes = [out_shape_h,
                      jax.ShapeDtypeStruct((B, _N, _N), jnp.bfloat16)]

        def body(h_ref, adj_ref, w_ref, as_ref, an_ref, b_ref, out_ref,
                 adjb_ref, feat_scr, ext_scr, en_scr, fnmax_scr):
            _layer_body(h_ref, adj_ref, w_ref, as_ref, an_ref, b_ref,
                        out_ref, adjb_ref, feat_scr, ext_scr, en_scr,
                        fnmax_scr, first_layer=True)
    else:
        out_specs = [out_spec_h]
        out_shapes = [out_shape_h]

        def body(h_ref, adj_ref, w_ref, as_ref, an_ref, b_ref, out_ref,
                 feat_scr, ext_scr, en_scr, fnmax_scr):
            _layer_body(h_ref, adj_ref, w_ref, as_ref, an_ref, b_ref,
                        out_ref, None, feat_scr, ext_scr, en_scr,
                        fnmax_scr, first_layer=False)

    return pl.pallas_call(
        body,
        grid=grid,
        in_specs=in_specs,
        out_specs=out_specs,
        out_shape=out_shapes,
        scratch_shapes=[
            pltpu.VMEM((_H, _N, _F_OUT), jnp.float32),
            pltpu.VMEM((_H, _N, 2 * _F_OUT), jnp.bfloat16),
            pltpu.VMEM((_H, 2, _N), jnp.bfloat16),
            pltpu.SMEM((_H,), jnp.float32),
        ],
    )(h, adj, Wl, asl[:, :, None], anl[:, None, :], bl[:, None, :])


def kernel(x, adj, W, a_self, a_neigh, b):
    h, adjb = _run_layer(x, adj, W[0], a_self[0], a_neigh[0], b[0],
                         first_layer=True)
    for l in range(1, _L):
        (h,) = _run_layer(h, adjb, W[l], a_self[l], a_neigh[l], b[l],
                          first_layer=False)
    return h
